# baseline (speedup 1.0000x reference)
import functools

import jax
import jax.numpy as jnp
from jax import lax
from jax.experimental import pallas as pl
from jax.experimental.pallas import tpu as pltpu
from jax.experimental.pallas import tpu_sc as plsc

MARGIN = 1.0
EPS = 1e-8
CHUNK = 128


@functools.lru_cache(maxsize=None)
def _make_sc_gather(B, D):
    info = plsc.get_sparse_core_info()
    NC, NS = info.num_cores, info.num_subcores
    NW = NC * NS
    b_per_w = B // NW
    assert B % NW == 0 and b_per_w % CHUNK == 0
    n_ch = b_per_w // CHUNK
    mesh = plsc.VectorSubcoreMesh(core_axis_name="c", subcore_axis_name="s")

    @functools.partial(
        pl.kernel,
        mesh=mesh,
        compiler_params=pltpu.CompilerParams(use_tc_tiling_on_sc=False),
        out_type=jax.ShapeDtypeStruct((6, B, D), jnp.float32),
        scratch_types=[
            pltpu.VMEM((n_ch, CHUNK), jnp.int32),
            pltpu.VMEM((CHUNK, D), jnp.float32),
            pltpu.VMEM((CHUNK, D), jnp.float32),
            pltpu.SemaphoreType.DMA,
            pltpu.SemaphoreType.DMA,
        ],
    )
    def sc_gather(ph_hbm, pr_hbm, pt_hbm, nh_hbm, nr_hbm, nt_hbm,
                  ent_hbm, rel_hbm, out_hbm,
                  idx_v, buf0, buf1, sem0, sem1):
        wid = lax.axis_index("s") * NC + lax.axis_index("c")
        base = wid * b_per_w
        specs = [
            (ph_hbm, ent_hbm),
            (pr_hbm, rel_hbm),
            (pt_hbm, ent_hbm),
            (nh_hbm, ent_hbm),
            (nr_hbm, rel_hbm),
            (nt_hbm, ent_hbm),
        ]
        bufs = (buf0, buf1)
        sems = (sem0, sem1)
        for g, (ids, table) in enumerate(specs):
            for j in range(n_ch):
                pltpu.sync_copy(ids.at[pl.ds(base + j * CHUNK, CHUNK)],
                                idx_v.at[j])
            for j in range(n_ch):
                k = j % 2
                pltpu.async_copy(table.at[idx_v.at[j]], bufs[k], sems[k]).wait()
                pltpu.sync_copy(bufs[k],
                                out_hbm.at[g, pl.ds(base + j * CHUNK, CHUNK)])

    return sc_gather


@functools.lru_cache(maxsize=None)
def _make_tc_loss(B, D, bs):
    ng = B // bs
    inv_b = 1.0 / B

    def body(e_ref, out_ref):
        i = pl.program_id(0)

        @pl.when(i == 0)
        def _():
            out_ref[0, 0] = 0.0

        e = e_ref[...]

        def norm_rows(x):
            n = jnp.sqrt(jnp.sum(x * x, axis=-1, keepdims=True))
            return x / jnp.maximum(n, 1e-12)

        h = norm_rows(e[0] + EPS)
        t = norm_rows(e[2] + EPS)
        h2 = norm_rows(e[3] + EPS)
        t2 = norm_rows(e[5] + EPS)
        pos = jnp.sqrt(jnp.sum((h + e[1] - t) ** 2, axis=-1))
        neg = jnp.sqrt(jnp.sum((h2 + e[4] - t2) ** 2, axis=-1))
        s = jnp.sum(jnp.maximum(0.0, MARGIN + pos - neg))
        out_ref[0, 0] += s

        @pl.when(i == ng - 1)
        def _():
            out_ref[0, 0] = out_ref[0, 0] * inv_b

    return pl.pallas_call(
        body,
        grid=(ng,),
        in_specs=[pl.BlockSpec((6, bs, D), lambda i: (0, i, 0))],
        out_specs=pl.BlockSpec(memory_space=pltpu.SMEM),
        out_shape=jax.ShapeDtypeStruct((1, 1), jnp.float32),
    )


def kernel(positive_triples, negative_triples, entity_embeddings,
           relation_embeddings):
    B = positive_triples.shape[1]
    D = entity_embeddings.shape[1]
    gathered = _make_sc_gather(B, D)(
        positive_triples[0], positive_triples[1], positive_triples[2],
        negative_triples[0], negative_triples[1], negative_triples[2],
        entity_embeddings, relation_embeddings)
    tot = _make_tc_loss(B, D, 1024)(gathered)
    return tot[0, 0]


# speedup vs baseline: 1.0761x; 1.0761x over previous
import functools

import jax
import jax.numpy as jnp
from jax import lax
from jax.experimental import pallas as pl
from jax.experimental.pallas import tpu as pltpu
from jax.experimental.pallas import tpu_sc as plsc

MARGIN = 1.0
EPS = 1e-8
CHUNK = 128


def _nrsqrt(x):
    xi = plsc.bitcast(x, jnp.int32)
    yi = jnp.int32(0x5F3759DF) - lax.shift_right_logical(xi, 1)
    y = plsc.bitcast(yi, jnp.float32)
    for _ in range(3):
        y = y * (1.5 - 0.5 * x * y * y)
    return y


@functools.lru_cache(maxsize=None)
def _make_sc_fused(B, D):
    info = plsc.get_sparse_core_info()
    NC, NS, L = info.num_cores, info.num_subcores, info.num_lanes
    NW = NC * NS
    b_per_w = B // NW
    assert B % NW == 0 and b_per_w % CHUNK == 0 and CHUNK % L == 0
    n_ch = b_per_w // CHUNK
    n_g = CHUNK // L
    mesh = plsc.VectorSubcoreMesh(core_axis_name="c", subcore_axis_name="s")

    row_scratch = [pltpu.VMEM((CHUNK, D), jnp.float32) for _ in range(12)]
    idx_scratch = [pltpu.VMEM((b_per_w,), jnp.int32) for _ in range(6)]

    @functools.partial(
        pl.kernel,
        mesh=mesh,
        compiler_params=pltpu.CompilerParams(
            use_tc_tiling_on_sc=False, needs_layout_passes=False),
        out_type=jax.ShapeDtypeStruct((NW, L), jnp.float32),
        scratch_types=idx_scratch + row_scratch + [
            pltpu.VMEM((12 * CHUNK,), jnp.float32),
            pltpu.VMEM((L,), jnp.float32),
            pltpu.SemaphoreType.DMA,
            pltpu.SemaphoreType.DMA,
        ],
    )
    def sc_fused(ph_hbm, pr_hbm, pt_hbm, nh_hbm, nr_hbm, nt_hbm,
                 ent_hbm, rel_hbm, out_hbm, *refs):
        idx_bufs = refs[0:6]
        bufs = (refs[6:12], refs[12:18])
        sums = refs[18]
        part_v = refs[19]
        sems = (refs[20], refs[21])
        wid = lax.axis_index("s") * NC + lax.axis_index("c")
        base = wid * b_per_w
        tables = (ent_hbm, rel_hbm, ent_hbm, ent_hbm, rel_hbm, ent_hbm)
        id_hbm = (ph_hbm, pr_hbm, pt_hbm, nh_hbm, nr_hbm, nt_hbm)

        for e in range(6):
            pltpu.sync_copy(id_hbm[e].at[pl.ds(base, b_per_w)], idx_bufs[e])

        def fire(c, p):
            return [
                pltpu.async_copy(
                    tables[e].at[idx_bufs[e].at[pl.ds(c * CHUNK, CHUNK)]],
                    bufs[p][e], sems[p])
                for e in range(6)
            ]

        nk = D // L
        last = lax.iota(jnp.int32, L) == (L - 1)

        def compute(p, sums, acc):
            def tbody(i, _):
                def ld(e):
                    return [bufs[p][e][i, pl.ds(k * L, L)] for k in range(nk)]

                h, r, t, h2, r2, t2 = (ld(e) for e in range(6))
                he = [v + EPS for v in h]
                te = [v + EPS for v in t]
                h2e = [v + EPS for v in h2]
                t2e = [v + EPS for v in t2]

                def red(a, b):
                    s = a[0] * b[0]
                    for k in range(1, nk):
                        s = s + a[k] * b[k]
                    return s

                terms = (red(he, he), red(te, te), red(r, r),
                         red(he, r), red(he, te), red(r, te),
                         red(h2e, h2e), red(t2e, t2e), red(r2, r2),
                         red(h2e, r2), red(h2e, t2e), red(r2, t2e))
                iv = jnp.full((L,), 0, jnp.int32) + i
                for m, v in enumerate(terms):
                    plsc.store_scatter(sums, [iv + m * CHUNK],
                                       plsc.cumsum(v), mask=last)
                return 0

            lax.fori_loop(0, CHUNK, tbody, 0)

            def gbody(g, acc):
                o = g * L
                (sh, st, sr, chr_, cht, crt,
                 sh2, st2, sr2, chr2, cht2, crt2) = (
                     sums[pl.ds(o + m * CHUNK, L)] for m in range(12))

                def score(sa, sb, sc_, cab, cac, cbc):
                    al = _nrsqrt(jnp.maximum(sa, 1e-24))
                    be = _nrsqrt(jnp.maximum(sb, 1e-24))
                    sq = (al * al * sa + be * be * sb + sc_
                          + 2.0 * al * cab - 2.0 * al * be * cac
                          - 2.0 * be * cbc)
                    sq = jnp.maximum(sq, 0.0)
                    return sq * _nrsqrt(jnp.maximum(sq, 1e-24))

                pos = score(sh, st, sr, chr_, cht, crt)
                neg = score(sh2, st2, sr2, chr2, cht2, crt2)
                return acc + jnp.maximum(0.0, MARGIN + pos - neg)

            return lax.fori_loop(0, n_g, gbody, acc)

        acc = jnp.zeros((L,), jnp.float32)
        pend = fire(0, 0)
        for c in range(n_ch):
            p = c % 2
            for d_ in pend:
                d_.wait()
            if c + 1 < n_ch:
                pend = fire(c + 1, 1 - p)
            acc = compute(p, sums, acc)

        part_v[...] = acc
        pltpu.sync_copy(part_v, out_hbm.at[wid])

    return sc_fused


@functools.lru_cache(maxsize=None)
def _make_tc_mean(NW, L, B):
    inv_b = 1.0 / B

    def body(x_ref, out_ref):
        out_ref[0, 0] = jnp.sum(x_ref[...]) * inv_b

    return pl.pallas_call(
        body,
        in_specs=[pl.BlockSpec(memory_space=pltpu.VMEM)],
        out_specs=pl.BlockSpec(memory_space=pltpu.SMEM),
        out_shape=jax.ShapeDtypeStruct((1, 1), jnp.float32),
    )


def kernel(positive_triples, negative_triples, entity_embeddings,
           relation_embeddings):
    B = positive_triples.shape[1]
    D = entity_embeddings.shape[1]
    parts = _make_sc_fused(B, D)(
        positive_triples[0], positive_triples[1], positive_triples[2],
        negative_triples[0], negative_triples[1], negative_triples[2],
        entity_embeddings, relation_embeddings)
    tot = _make_tc_mean(parts.shape[0], parts.shape[1], B)(parts)
    return tot[0, 0]
